# baseline (speedup 1.0000x reference)
import functools

import jax
import jax.numpy as jnp
from jax import lax
from jax.experimental import pallas as pl
from jax.experimental.pallas import tpu as pltpu
from jax.experimental.pallas import tpu_sc as plsc

_K1 = 1.5
_B = 0.75
_N = 50000
_V = 1000
_Q = 32
_L = 16
_TOPK = 10

_BN = 1000
_NBLK = _N // _BN
_BN2 = 2048
_NBLK2 = -(-_N // _BN2)
_CHUNKS = _N // 16


def _df_body(tf_ref, df_ref, acc_ref):
    i = pl.program_id(0)

    @pl.when(i == 0)
    def _():
        acc_ref[...] = jnp.zeros_like(acc_ref)

    tfb = tf_ref[...]
    acc_ref[...] += jnp.sum((tfb > 0).astype(jnp.float32), axis=0, keepdims=True)

    @pl.when(i == _NBLK - 1)
    def _():
        df_ref[...] = acc_ref[...]


def _df_pass(tf):
    return pl.pallas_call(
        _df_body,
        grid=(_NBLK,),
        in_specs=[pl.BlockSpec((_BN, _V), lambda i: (i, 0))],
        out_specs=pl.BlockSpec((1, _V), lambda i: (0, 0)),
        out_shape=jax.ShapeDtypeStruct((1, _V), jnp.float32),
        scratch_shapes=[pltpu.VMEM((1, _V), jnp.float32)],
        compiler_params=pltpu.CompilerParams(
            dimension_semantics=("arbitrary",)),
    )(tf)


_NBUF = 5
_BR = 1000
_NSTEP = _N // _BR


def _df_body_manual(tf_hbm, df_ref, buf, acc_ref, sems):
    for b in range(_NBUF):
        pltpu.make_async_copy(
            tf_hbm.at[pl.ds(b * _BR, _BR), :], buf.at[b], sems.at[b]).start()
    acc_ref[...] = jnp.zeros_like(acc_ref)

    def outer(i0, _):
        for b in range(_NBUF):
            i = i0 * _NBUF + b
            pltpu.make_async_copy(
                tf_hbm.at[pl.ds(i * _BR, _BR), :], buf.at[b], sems.at[b]
            ).wait()
            blk = buf[b, 0:1, :]
            acc_ref[...] += blk

            @pl.when(i + _NBUF < _NSTEP)
            def _():
                pltpu.make_async_copy(
                    tf_hbm.at[pl.ds((i + _NBUF) * _BR, _BR), :], buf.at[b],
                    sems.at[b]).start()
        return 0

    lax.fori_loop(0, _NSTEP // _NBUF, outer, 0)
    df_ref[...] = acc_ref[...]


def _df_pass_manual(tf):
    return pl.pallas_call(
        _df_body_manual,
        in_specs=[pl.BlockSpec(memory_space=pl.ANY)],
        out_specs=pl.BlockSpec(memory_space=pltpu.MemorySpace.VMEM),
        out_shape=jax.ShapeDtypeStruct((1, _V), jnp.float32),
        scratch_shapes=[
            pltpu.VMEM((_NBUF, _BR, _V), jnp.float32),
            pltpu.VMEM((1, _V), jnp.float32),
            pltpu.SemaphoreType.DMA((_NBUF,)),
        ],
    )(tf)


def _score_body(tf_ref, dl_ref, idf_ref, cnt_ref, avg_ref, out_ref):
    tfb = tf_ref[...]
    dl = dl_ref[...]
    avg = avg_ref[0, 0]
    norm = _K1 * (1.0 - _B + _B * dl / avg)
    num = tfb * (_K1 + 1.0)
    aprime = idf_ref[...] * num / (tfb + norm)
    out_ref[...] = lax.dot_general(
        cnt_ref[...], aprime,
        (((1,), (1,)), ((), ())),
        preferred_element_type=jnp.float32,
        precision=lax.Precision.HIGHEST)


def _score_pass(tf, dl2d, idf, counts, avg):
    return pl.pallas_call(
        _score_body,
        grid=(_NBLK2,),
        in_specs=[
            pl.BlockSpec((_BN2, _V), lambda i: (i, 0)),
            pl.BlockSpec((_BN2, 1), lambda i: (i, 0)),
            pl.BlockSpec((1, _V), lambda i: (0, 0)),
            pl.BlockSpec((_Q, _V), lambda i: (0, 0)),
            pl.BlockSpec((1, 1), lambda i: (0, 0)),
        ],
        out_specs=pl.BlockSpec((_Q, _BN2), lambda i: (0, i)),
        out_shape=jax.ShapeDtypeStruct((_Q, _N), jnp.float32),
        compiler_params=pltpu.CompilerParams(
            dimension_semantics=("arbitrary",)),
    )(tf, dl2d, idf, counts, avg)


_NBUF2 = 4
_BR2 = 1024
_NCH2 = _N // _BR2
_TAIL2 = _N - _NCH2 * _BR2


def _score_chunk(tfb, dl, idf, cnt, avg):
    norm = _K1 * (1.0 - _B + _B * dl / avg)
    num = tfb * (_K1 + 1.0)
    aprime = idf * num / (tfb + norm)
    return lax.dot_general(
        cnt, aprime, (((1,), (1,)), ((), ())),
        preferred_element_type=jnp.float32,
        precision=lax.Precision.HIGHEST)


def _score_body_manual(tf_hbm, dl_ref, idf_ref, cnt_ref, avg_ref, out_ref,
                       buf, sems):
    for b in range(_NBUF2):
        pltpu.make_async_copy(
            tf_hbm.at[pl.ds(b * _BR2, _BR2), :], buf.at[b], sems.at[b]).start()
    avg = avg_ref[0, 0]
    idf = idf_ref[...]
    cnt = cnt_ref[...]

    def outer(i0, _):
        for b in range(_NBUF2):
            i = i0 * _NBUF2 + b
            pltpu.make_async_copy(
                tf_hbm.at[pl.ds(i * _BR2, _BR2), :], buf.at[b], sems.at[b]
            ).wait()
            tfb = buf[b]
            dl = dl_ref[pl.ds(i * _BR2, _BR2), :]
            sc = _score_chunk(tfb, dl, idf, cnt, avg)
            out_ref[:, pl.ds(i * _BR2, _BR2)] = sc

            nxt = i + _NBUF2

            @pl.when(nxt < _NCH2)
            def _():
                pltpu.make_async_copy(
                    tf_hbm.at[pl.ds(nxt * _BR2, _BR2), :], buf.at[b],
                    sems.at[b]).start()
        return 0

    lax.fori_loop(0, _NCH2 // _NBUF2, outer, 0)

    base = _NCH2 * _BR2
    pltpu.make_async_copy(
        tf_hbm.at[pl.ds(base, _TAIL2), :], buf.at[0, pl.ds(0, _TAIL2)],
        sems.at[0]).start()
    pltpu.make_async_copy(
        tf_hbm.at[pl.ds(base, _TAIL2), :], buf.at[0, pl.ds(0, _TAIL2)],
        sems.at[0]).wait()
    tfb = buf[0, pl.ds(0, _TAIL2)]
    dl = dl_ref[pl.ds(base, _TAIL2), :]
    out_ref[:, pl.ds(base, _TAIL2)] = _score_chunk(tfb, dl, idf, cnt, avg)


def _score_pass_manual(tf, dl2d, idf, counts, avg):
    return pl.pallas_call(
        _score_body_manual,
        in_specs=[
            pl.BlockSpec(memory_space=pl.ANY),
            pl.BlockSpec(memory_space=pltpu.MemorySpace.VMEM),
            pl.BlockSpec(memory_space=pltpu.MemorySpace.VMEM),
            pl.BlockSpec(memory_space=pltpu.MemorySpace.VMEM),
            pl.BlockSpec(memory_space=pltpu.MemorySpace.VMEM),
        ],
        out_specs=pl.BlockSpec(memory_space=pltpu.MemorySpace.VMEM),
        out_shape=jax.ShapeDtypeStruct((_Q, _N), jnp.float32),
        scratch_shapes=[
            pltpu.VMEM((_NBUF2, _BR2, _V), jnp.float32),
            pltpu.SemaphoreType.DMA((_NBUF2,)),
        ],
    )(tf, dl2d, idf, counts, avg)


_CB = 128
_NCB = -(-_V // _CB)
_RCH = 2048
_NRCH = _N // _RCH


_NCBF = 7
_TW = _V - _NCBF * _CB
_TOFF = _NCBF * _CB
_TAILR = _N - _NRCH * _RCH
_TNB = 3


def _fused_body(tf_hbm, dl_ref, cnt_ref, avg_ref, out_ref, bufa, bufb,
                tring, sems, tsems):
    bufs = [bufa, bufb]
    pltpu.make_async_copy(
        tf_hbm.at[:, pl.ds(0, _CB)], bufs[0], sems.at[0]).start()
    avg = avg_ref[0, 0]

    def norm_col(r0, rn):
        dlr = dl_ref[0:1, pl.ds(r0, rn)]
        return (_K1 * (1.0 - _B + _B * dlr / avg)).T

    for c in range(_NCBF):
        cur = bufs[c % 2]
        pltpu.make_async_copy(
            tf_hbm.at[:, pl.ds(c * _CB, _CB)], cur, sems.at[c % 2]).wait()
        if c + 1 < _NCBF:
            pltpu.make_async_copy(
                tf_hbm.at[:, pl.ds((c + 1) * _CB, _CB)], bufs[(c + 1) % 2],
                sems.at[(c + 1) % 2]).start()

        def df_chunk(i, d):
            blk = cur[pl.ds(i * _RCH, _RCH), :]
            return d + jnp.sum((blk > 0).astype(jnp.float32), axis=0,
                               keepdims=True)

        df = lax.fori_loop(0, _NRCH, df_chunk,
                           jnp.zeros((1, _CB), jnp.float32))
        blk = cur[pl.ds(_NRCH * _RCH, _TAILR), :]
        df += jnp.sum((blk > 0).astype(jnp.float32), axis=0, keepdims=True)

        idf = jnp.log((_N - df + 0.5) / (df + 0.5))
        cnt = cnt_ref[c]

        def sc_chunk(r0, rn):
            tfb = cur[pl.ds(r0, rn), :]
            num = tfb * (_K1 + 1.0)
            ap = idf * num / (tfb + norm_col(r0, rn))
            return lax.dot_general(
                cnt, ap, (((1,), (1,)), ((), ())),
                preferred_element_type=jnp.float32,
                precision=lax.Precision.HIGHEST)

        if c == 0:
            def body0(i, _):
                r0 = i * _RCH
                out_ref[:, pl.ds(r0, _RCH)] = sc_chunk(r0, _RCH)
                return 0
            lax.fori_loop(0, _NRCH, body0, 0)
            out_ref[:, pl.ds(_NRCH * _RCH, _TAILR)] = sc_chunk(
                _NRCH * _RCH, _TAILR)
        else:
            def bodyn(i, _):
                r0 = i * _RCH
                out_ref[:, pl.ds(r0, _RCH)] += sc_chunk(r0, _RCH)
                return 0
            lax.fori_loop(0, _NRCH, bodyn, 0)
            out_ref[:, pl.ds(_NRCH * _RCH, _TAILR)] += sc_chunk(
                _NRCH * _RCH, _TAILR)

    def tcopy(i, b):
        return pltpu.make_async_copy(
            tf_hbm.at[pl.ds(i * _RCH, _RCH), pl.ds(_TOFF, _TW)],
            tring.at[b], tsems.at[b])

    for b in range(_TNB):
        tcopy(b, b).start()

    def tdf_outer(i0, d):
        for b in range(_TNB):
            i = i0 * _TNB + b
            tcopy(i, b).wait()
            blk = tring[b]
            d = d + jnp.sum((blk > 0).astype(jnp.float32), axis=0,
                            keepdims=True)

            @pl.when(i + _TNB < _NRCH)
            def _():
                tcopy(i + _TNB, b).start()
        return d

    dft = lax.fori_loop(0, _NRCH // _TNB, tdf_outer,
                        jnp.zeros((1, _TW), jnp.float32))
    pltpu.make_async_copy(
        tf_hbm.at[pl.ds(_NRCH * _RCH, _TAILR), pl.ds(_TOFF, _TW)],
        tring.at[0, pl.ds(0, _TAILR)], tsems.at[0]).start()
    pltpu.make_async_copy(
        tf_hbm.at[pl.ds(_NRCH * _RCH, _TAILR), pl.ds(_TOFF, _TW)],
        tring.at[0, pl.ds(0, _TAILR)], tsems.at[0]).wait()
    blk = tring[0, pl.ds(0, _TAILR)]
    dft += jnp.sum((blk > 0).astype(jnp.float32), axis=0, keepdims=True)

    idft = jnp.log((_N - dft + 0.5) / (dft + 0.5))
    cntt = cnt_ref[_NCBF][:, 0:_TW]

    def tsc(tfb, r0, rn):
        num = tfb * (_K1 + 1.0)
        ap = idft * num / (tfb + norm_col(r0, rn))
        return lax.dot_general(
            cntt, ap, (((1,), (1,)), ((), ())),
            preferred_element_type=jnp.float32,
            precision=lax.Precision.HIGHEST)

    for b in range(_TNB):
        tcopy(b, b).start()

    def tsc_outer(i0, _):
        for b in range(_TNB):
            i = i0 * _TNB + b
            tcopy(i, b).wait()
            r0 = i * _RCH
            out_ref[:, pl.ds(r0, _RCH)] += tsc(tring[b], r0, _RCH)

            @pl.when(i + _TNB < _NRCH)
            def _():
                tcopy(i + _TNB, b).start()
        return 0

    lax.fori_loop(0, _NRCH // _TNB, tsc_outer, 0)
    pltpu.make_async_copy(
        tf_hbm.at[pl.ds(_NRCH * _RCH, _TAILR), pl.ds(_TOFF, _TW)],
        tring.at[0, pl.ds(0, _TAILR)], tsems.at[0]).start()
    pltpu.make_async_copy(
        tf_hbm.at[pl.ds(_NRCH * _RCH, _TAILR), pl.ds(_TOFF, _TW)],
        tring.at[0, pl.ds(0, _TAILR)], tsems.at[0]).wait()
    out_ref[:, pl.ds(_NRCH * _RCH, _TAILR)] += tsc(
        tring[0, pl.ds(0, _TAILR)], _NRCH * _RCH, _TAILR)


def _fused_pass(tf, dl2d, counts_blocks, avg):
    return pl.pallas_call(
        _fused_body,
        in_specs=[
            pl.BlockSpec(memory_space=pl.ANY),
            pl.BlockSpec(memory_space=pltpu.MemorySpace.VMEM),
            pl.BlockSpec(memory_space=pltpu.MemorySpace.VMEM),
            pl.BlockSpec(memory_space=pltpu.MemorySpace.VMEM),
        ],
        out_specs=pl.BlockSpec(memory_space=pltpu.MemorySpace.VMEM),
        out_shape=jax.ShapeDtypeStruct((_Q, _N), jnp.float32),
        scratch_shapes=[
            pltpu.VMEM((_N, _CB), jnp.float32),
            pltpu.VMEM((_N, _CB), jnp.float32),
            pltpu.VMEM((_TNB, _RCH, _TW), jnp.float32),
            pltpu.SemaphoreType.DMA((2,)),
            pltpu.SemaphoreType.DMA((_TNB,)),
        ],
        compiler_params=pltpu.CompilerParams(
            vmem_limit_bytes=63 * 1024 * 1024),
    )(tf, dl2d, counts_blocks, avg)


_GATHER_DNUMS = lax.GatherDimensionNumbers(
    offset_dims=(), collapsed_slice_dims=(0,), start_index_map=(0,))


def _lane_permute(x, idx):
    return lax.gather(x, idx[:, None], _GATHER_DNUMS, slice_sizes=(1,),
                      mode=lax.GatherScatterMode.PROMISE_IN_BOUNDS)

def _topk_body(scores_ref, vals_ref, idx_ref, buf, vv, vi):
    c = lax.axis_index("c")
    s = lax.axis_index("s")
    q = c * 16 + s

    pltpu.sync_copy(scores_ref.at[q], buf)

    neg = jnp.float32(-jnp.inf)
    lanes = lax.iota(jnp.int32, 16)
    big = jnp.int32(2**31 - 1)

    outv = jnp.zeros((16,), jnp.float32)
    outi = jnp.zeros((16,), jnp.int32)

    for kk in range(_TOPK):
        def body(i, carry):
            m, mi = carry
            v = buf[pl.ds(i * 16, 16)]
            upd = v > m
            m = jnp.where(upd, v, m)
            mi = jnp.where(upd, i, mi)
            return m, mi

        m, mi = lax.fori_loop(
            0, _CHUNKS, body,
            (jnp.full((16,), neg, jnp.float32), jnp.zeros((16,), jnp.int32)),
            unroll=8)
        mx = m
        for sh in (8, 4, 2, 1):
            mx = jnp.maximum(mx, _lane_permute(mx, lanes ^ sh))
        cand = jnp.where(m == mx, mi * 16 + lanes, big)
        pos = cand
        for sh in (8, 4, 2, 1):
            pos = jnp.minimum(pos, _lane_permute(pos, lanes ^ sh))
        outv = jnp.where(lanes == kk, mx, outv)
        outi = jnp.where(lanes == kk, pos, outi)
        plsc.store_scatter(buf, [pos], jnp.full((16,), neg, jnp.float32),
                           mask=lanes == 0)

    vv[...] = outv
    vi[...] = outi
    pltpu.sync_copy(vv, vals_ref.at[q])
    pltpu.sync_copy(vi, idx_ref.at[q])


def _topk_pass(scores_t):
    mesh = plsc.VectorSubcoreMesh(core_axis_name="c", subcore_axis_name="s")
    call = functools.partial(
        pl.kernel,
        out_type=[
            jax.ShapeDtypeStruct((_Q, 16), jnp.float32),
            jax.ShapeDtypeStruct((_Q, 16), jnp.int32),
        ],
        mesh=mesh,
        scratch_types=[
            pltpu.VMEM((_N,), jnp.float32),
            pltpu.VMEM((16,), jnp.float32),
            pltpu.VMEM((16,), jnp.int32),
        ],
        compiler_params=pltpu.CompilerParams(needs_layout_passes=False),
    )(_topk_body)
    return call(scores_t)


_NG = 3125
_GCH = 196


def _topk_body2(scores_ref, vals_ref, idx_ref, buf, gbuf, cval, cidx, vv, vi):
    c = lax.axis_index("c")
    s = lax.axis_index("s")
    q = c * 16 + s

    pltpu.sync_copy(scores_ref.at[q], buf)

    neg = jnp.float32(-jnp.inf)
    lanes = lax.iota(jnp.int32, 16)
    big = jnp.int32(2**31 - 1)

    def gbody(cc, carry):
        g0 = cc * 16
        m = jnp.full((16,), neg, jnp.float32)
        for j in range(16):
            m = jnp.maximum(m, buf[pl.ds(g0 + j * _NG, 16)])
        gbuf[pl.ds(g0, 16)] = m
        return carry

    lax.fori_loop(0, _GCH - 1, gbody, 0, unroll=4)
    g0 = (_GCH - 1) * 16
    m = jnp.full((16,), neg, jnp.float32)
    for j in range(15):
        m = jnp.maximum(m, buf[pl.ds(g0 + j * _NG, 16)])
    v15 = buf[pl.ds(_N - 16, 16)]
    shift = g0 + 15 * _NG - (_N - 16)
    m15 = _lane_permute(v15, jnp.minimum(lanes + shift, 15))
    m = jnp.maximum(m, m15)
    gbuf[pl.ds(g0, 16)] = jnp.where(g0 + lanes < _NG, m, neg)

    for kk in range(_TOPK):
        def body(i, carry):
            mm, mi = carry
            v = gbuf[pl.ds(i * 16, 16)]
            upd = v > mm
            mm = jnp.where(upd, v, mm)
            mi = jnp.where(upd, i, mi)
            return mm, mi

        mm, mi = lax.fori_loop(
            0, _GCH, body,
            (jnp.full((16,), neg, jnp.float32), jnp.zeros((16,), jnp.int32)),
            unroll=8)
        mx = mm
        for sh in (8, 4, 2, 1):
            mx = jnp.maximum(mx, _lane_permute(mx, lanes ^ sh))
        cand = jnp.where(mm == mx, mi * 16 + lanes, big)
        gsel = cand
        for sh in (8, 4, 2, 1):
            gsel = jnp.minimum(gsel, _lane_permute(gsel, lanes ^ sh))
        plsc.store_scatter(gbuf, [gsel], jnp.full((16,), neg, jnp.float32),
                           mask=lanes == 0)
        didx = gsel + _NG * lanes
        cval[pl.ds(kk * 16, 16)] = plsc.load_gather(buf, [didx])
        cidx[pl.ds(kk * 16, 16)] = didx

    outv = jnp.zeros((16,), jnp.float32)
    outi = jnp.zeros((16,), jnp.int32)
    for kk in range(_TOPK):
        mm = jnp.full((16,), neg, jnp.float32)
        mi = jnp.zeros((16,), jnp.int32)
        for i in range(_TOPK):
            v = cval[pl.ds(i * 16, 16)]
            upd = v > mm
            mm = jnp.where(upd, v, mm)
            mi = jnp.where(upd, i, mi)
        mx = mm
        for sh in (8, 4, 2, 1):
            mx = jnp.maximum(mx, _lane_permute(mx, lanes ^ sh))
        cand = jnp.where(mm == mx, mi * 16 + lanes, big)
        pos = cand
        for sh in (8, 4, 2, 1):
            pos = jnp.minimum(pos, _lane_permute(pos, lanes ^ sh))
        dsel = plsc.load_gather(cidx, [pos])
        outv = jnp.where(lanes == kk, mx, outv)
        outi = jnp.where(lanes == kk, dsel, outi)
        plsc.store_scatter(cval, [pos], jnp.full((16,), neg, jnp.float32),
                           mask=lanes == 0)

    vv[...] = outv
    vi[...] = outi
    pltpu.sync_copy(vv, vals_ref.at[q])
    pltpu.sync_copy(vi, idx_ref.at[q])


def _topk_pass2(scores_t):
    mesh = plsc.VectorSubcoreMesh(core_axis_name="c", subcore_axis_name="s")
    call = functools.partial(
        pl.kernel,
        out_type=[
            jax.ShapeDtypeStruct((_Q, 16), jnp.float32),
            jax.ShapeDtypeStruct((_Q, 16), jnp.int32),
        ],
        mesh=mesh,
        scratch_types=[
            pltpu.VMEM((_N,), jnp.float32),
            pltpu.VMEM((_GCH * 16,), jnp.float32),
            pltpu.VMEM((_TOPK * 16,), jnp.float32),
            pltpu.VMEM((_TOPK * 16,), jnp.int32),
            pltpu.VMEM((16,), jnp.float32),
            pltpu.VMEM((16,), jnp.int32),
        ],
        compiler_params=pltpu.CompilerParams(needs_layout_passes=False),
    )(_topk_body2)
    return call(scores_t)


def kernel(tf, doc_len, query_terms, k):
    doc_len = doc_len.astype(jnp.float32)
    tf = tf.astype(jnp.float32)

    counts = jnp.sum(
        jax.nn.one_hot(query_terms, _V, dtype=jnp.float32), axis=1)

    avg = jnp.mean(doc_len).reshape(1, 1)
    dl_row = doc_len.reshape(1, _N)

    cbs = [counts[:, c * _CB:(c + 1) * _CB] for c in range(_NCBF)]
    cbs.append(jnp.pad(counts[:, _TOFF:], ((0, 0), (0, _CB - _TW))))
    counts_blocks = jnp.stack(cbs)

    scores_t = _fused_pass(tf, dl_row, counts_blocks, avg)

    vals_p, idx_p = _topk_pass2(scores_t)
    vals = vals_p[:, :_TOPK]
    idx = idx_p[:, :_TOPK]
    vals = vals + 0.0 * (jnp.asarray(k, jnp.float32) - float(_TOPK))
    return vals, idx


# speedup vs baseline: 2.5823x; 2.5823x over previous
import functools

import jax
import jax.numpy as jnp
from jax import lax
from jax.experimental import pallas as pl
from jax.experimental.pallas import tpu as pltpu
from jax.experimental.pallas import tpu_sc as plsc

_K1 = 1.5
_B = 0.75
_N = 50000
_V = 1000
_Q = 32
_L = 16
_TOPK = 10

_BN = 1000
_NBLK = _N // _BN
_BN2 = 2048
_NBLK2 = -(-_N // _BN2)
_CHUNKS = _N // 16


def _df_body(tf_ref, df_ref, acc_ref):
    i = pl.program_id(0)

    @pl.when(i == 0)
    def _():
        acc_ref[...] = jnp.zeros_like(acc_ref)

    tfb = tf_ref[...]
    acc_ref[...] += jnp.sum((tfb > 0).astype(jnp.float32), axis=0, keepdims=True)

    @pl.when(i == _NBLK - 1)
    def _():
        df_ref[...] = acc_ref[...]


def _df_pass(tf):
    return pl.pallas_call(
        _df_body,
        grid=(_NBLK,),
        in_specs=[pl.BlockSpec((_BN, _V), lambda i: (i, 0))],
        out_specs=pl.BlockSpec((1, _V), lambda i: (0, 0)),
        out_shape=jax.ShapeDtypeStruct((1, _V), jnp.float32),
        scratch_shapes=[pltpu.VMEM((1, _V), jnp.float32)],
        compiler_params=pltpu.CompilerParams(
            dimension_semantics=("arbitrary",)),
    )(tf)


_NBUF = 5
_BR = 1000
_NSTEP = _N // _BR


def _df_body_manual(tf_hbm, df_ref, buf, acc_ref, sems):
    for b in range(_NBUF):
        pltpu.make_async_copy(
            tf_hbm.at[pl.ds(b * _BR, _BR), :], buf.at[b], sems.at[b]).start()
    acc_ref[...] = jnp.zeros_like(acc_ref)

    def outer(i0, _):
        for b in range(_NBUF):
            i = i0 * _NBUF + b
            pltpu.make_async_copy(
                tf_hbm.at[pl.ds(i * _BR, _BR), :], buf.at[b], sems.at[b]
            ).wait()
            blk = buf[b, 0:1, :]
            acc_ref[...] += blk

            @pl.when(i + _NBUF < _NSTEP)
            def _():
                pltpu.make_async_copy(
                    tf_hbm.at[pl.ds((i + _NBUF) * _BR, _BR), :], buf.at[b],
                    sems.at[b]).start()
        return 0

    lax.fori_loop(0, _NSTEP // _NBUF, outer, 0)
    df_ref[...] = acc_ref[...]


def _df_pass_manual(tf):
    return pl.pallas_call(
        _df_body_manual,
        in_specs=[pl.BlockSpec(memory_space=pl.ANY)],
        out_specs=pl.BlockSpec(memory_space=pltpu.MemorySpace.VMEM),
        out_shape=jax.ShapeDtypeStruct((1, _V), jnp.float32),
        scratch_shapes=[
            pltpu.VMEM((_NBUF, _BR, _V), jnp.float32),
            pltpu.VMEM((1, _V), jnp.float32),
            pltpu.SemaphoreType.DMA((_NBUF,)),
        ],
    )(tf)


def _score_body(tf_ref, dl_ref, idf_ref, cnt_ref, avg_ref, out_ref):
    tfb = tf_ref[...]
    dl = dl_ref[...]
    avg = avg_ref[0, 0]
    norm = _K1 * (1.0 - _B + _B * dl / avg)
    num = tfb * (_K1 + 1.0)
    aprime = idf_ref[...] * num / (tfb + norm)
    out_ref[...] = lax.dot_general(
        cnt_ref[...], aprime,
        (((1,), (1,)), ((), ())),
        preferred_element_type=jnp.float32,
        precision=lax.Precision.HIGHEST)


def _score_pass(tf, dl2d, idf, counts, avg):
    return pl.pallas_call(
        _score_body,
        grid=(_NBLK2,),
        in_specs=[
            pl.BlockSpec((_BN2, _V), lambda i: (i, 0)),
            pl.BlockSpec((_BN2, 1), lambda i: (i, 0)),
            pl.BlockSpec((1, _V), lambda i: (0, 0)),
            pl.BlockSpec((_Q, _V), lambda i: (0, 0)),
            pl.BlockSpec((1, 1), lambda i: (0, 0)),
        ],
        out_specs=pl.BlockSpec((_Q, _BN2), lambda i: (0, i)),
        out_shape=jax.ShapeDtypeStruct((_Q, _N), jnp.float32),
        compiler_params=pltpu.CompilerParams(
            dimension_semantics=("arbitrary",)),
    )(tf, dl2d, idf, counts, avg)


_NBUF2 = 4
_BR2 = 1024
_NCH2 = _N // _BR2
_TAIL2 = _N - _NCH2 * _BR2


def _score_chunk(tfb, dl, idf, cnt, avg):
    norm = _K1 * (1.0 - _B + _B * dl / avg)
    num = tfb * (_K1 + 1.0)
    aprime = idf * num / (tfb + norm)
    return lax.dot_general(
        cnt, aprime, (((1,), (1,)), ((), ())),
        preferred_element_type=jnp.float32,
        precision=lax.Precision.HIGHEST)


def _score_body_manual(tf_hbm, dl_ref, idf_ref, cnt_ref, avg_ref, out_ref,
                       buf, sems):
    for b in range(_NBUF2):
        pltpu.make_async_copy(
            tf_hbm.at[pl.ds(b * _BR2, _BR2), :], buf.at[b], sems.at[b]).start()
    avg = avg_ref[0, 0]
    idf = idf_ref[...]
    cnt = cnt_ref[...]

    def outer(i0, _):
        for b in range(_NBUF2):
            i = i0 * _NBUF2 + b
            pltpu.make_async_copy(
                tf_hbm.at[pl.ds(i * _BR2, _BR2), :], buf.at[b], sems.at[b]
            ).wait()
            tfb = buf[b]
            dl = dl_ref[pl.ds(i * _BR2, _BR2), :]
            sc = _score_chunk(tfb, dl, idf, cnt, avg)
            out_ref[:, pl.ds(i * _BR2, _BR2)] = sc

            nxt = i + _NBUF2

            @pl.when(nxt < _NCH2)
            def _():
                pltpu.make_async_copy(
                    tf_hbm.at[pl.ds(nxt * _BR2, _BR2), :], buf.at[b],
                    sems.at[b]).start()
        return 0

    lax.fori_loop(0, _NCH2 // _NBUF2, outer, 0)

    base = _NCH2 * _BR2
    pltpu.make_async_copy(
        tf_hbm.at[pl.ds(base, _TAIL2), :], buf.at[0, pl.ds(0, _TAIL2)],
        sems.at[0]).start()
    pltpu.make_async_copy(
        tf_hbm.at[pl.ds(base, _TAIL2), :], buf.at[0, pl.ds(0, _TAIL2)],
        sems.at[0]).wait()
    tfb = buf[0, pl.ds(0, _TAIL2)]
    dl = dl_ref[pl.ds(base, _TAIL2), :]
    out_ref[:, pl.ds(base, _TAIL2)] = _score_chunk(tfb, dl, idf, cnt, avg)


def _score_pass_manual(tf, dl2d, idf, counts, avg):
    return pl.pallas_call(
        _score_body_manual,
        in_specs=[
            pl.BlockSpec(memory_space=pl.ANY),
            pl.BlockSpec(memory_space=pltpu.MemorySpace.VMEM),
            pl.BlockSpec(memory_space=pltpu.MemorySpace.VMEM),
            pl.BlockSpec(memory_space=pltpu.MemorySpace.VMEM),
            pl.BlockSpec(memory_space=pltpu.MemorySpace.VMEM),
        ],
        out_specs=pl.BlockSpec(memory_space=pltpu.MemorySpace.VMEM),
        out_shape=jax.ShapeDtypeStruct((_Q, _N), jnp.float32),
        scratch_shapes=[
            pltpu.VMEM((_NBUF2, _BR2, _V), jnp.float32),
            pltpu.SemaphoreType.DMA((_NBUF2,)),
        ],
    )(tf, dl2d, idf, counts, avg)


_CB = 128
_NCB = -(-_V // _CB)
_TBH = [_CB] * (_NCB - 1) + [_V - _CB * (_NCB - 1)]
_RCH = 2048
_NRCH = _N // _RCH
_TAILR = _N - _NRCH * _RCH


def _fused_body(tft_hbm, dl_ref, cnt_ref, avg_ref, out_ref, bufa, bufb, sems):
    bufs = [bufa, bufb]

    def bcopy(c, b):
        h = _TBH[c]
        src = tft_hbm.at[pl.ds(c * _CB, h), :]
        dst = bufs[b] if h == _CB else bufs[b].at[pl.ds(0, h), :]
        return pltpu.make_async_copy(src, dst, sems.at[b])

    bcopy(0, 0).start()
    avg = avg_ref[0, 0]

    for c in range(_NCB):
        h = _TBH[c]
        cur = bufs[c % 2]
        bcopy(c, c % 2).wait()
        if c + 1 < _NCB:
            bcopy(c + 1, (c + 1) % 2).start()

        def df_chunk(i, d):
            blk = cur[0:h, pl.ds(i * _RCH, _RCH)]
            return d + jnp.sum((blk > 0).astype(jnp.float32), axis=1,
                               keepdims=True)

        df = lax.fori_loop(0, _NRCH, df_chunk,
                           jnp.zeros((h, 1), jnp.float32))
        blk = cur[0:h, pl.ds(_NRCH * _RCH, _TAILR)]
        df += jnp.sum((blk > 0).astype(jnp.float32), axis=1, keepdims=True)

        idf = jnp.log((_N - df + 0.5) / (df + 0.5))
        cnt = cnt_ref[c][:, 0:h]

        def sc_chunk(r0, rn):
            tfb = cur[0:h, pl.ds(r0, rn)]
            dlr = dl_ref[0:1, pl.ds(r0, rn)]
            norm = _K1 * (1.0 - _B + _B * dlr / avg)
            num = tfb * (_K1 + 1.0)
            ap = idf * num / (tfb + norm)
            return lax.dot_general(
                cnt, ap, (((1,), (0,)), ((), ())),
                preferred_element_type=jnp.float32,
                precision=lax.Precision.HIGHEST)

        if c == 0:
            def body0(i, _):
                r0 = i * _RCH
                out_ref[:, pl.ds(r0, _RCH)] = sc_chunk(r0, _RCH)
                return 0
            lax.fori_loop(0, _NRCH, body0, 0)
            out_ref[:, pl.ds(_NRCH * _RCH, _TAILR)] = sc_chunk(
                _NRCH * _RCH, _TAILR)
        else:
            def bodyn(i, _):
                r0 = i * _RCH
                out_ref[:, pl.ds(r0, _RCH)] += sc_chunk(r0, _RCH)
                return 0
            lax.fori_loop(0, _NRCH, bodyn, 0)
            out_ref[:, pl.ds(_NRCH * _RCH, _TAILR)] += sc_chunk(
                _NRCH * _RCH, _TAILR)


def _fused_pass(tft, dl_row, counts_blocks, avg):
    return pl.pallas_call(
        _fused_body,
        in_specs=[
            pl.BlockSpec(memory_space=pl.ANY),
            pl.BlockSpec(memory_space=pltpu.MemorySpace.VMEM),
            pl.BlockSpec(memory_space=pltpu.MemorySpace.VMEM),
            pl.BlockSpec(memory_space=pltpu.MemorySpace.VMEM),
        ],
        out_specs=pl.BlockSpec(memory_space=pltpu.MemorySpace.VMEM),
        out_shape=jax.ShapeDtypeStruct((_Q, _N), jnp.float32),
        scratch_shapes=[
            pltpu.VMEM((_CB, _N), jnp.float32),
            pltpu.VMEM((_CB, _N), jnp.float32),
            pltpu.SemaphoreType.DMA((2,)),
        ],
        compiler_params=pltpu.CompilerParams(
            vmem_limit_bytes=63 * 1024 * 1024),
    )(tft, dl_row, counts_blocks, avg)


_GATHER_DNUMS = lax.GatherDimensionNumbers(
    offset_dims=(), collapsed_slice_dims=(0,), start_index_map=(0,))


def _lane_permute(x, idx):
    return lax.gather(x, idx[:, None], _GATHER_DNUMS, slice_sizes=(1,),
                      mode=lax.GatherScatterMode.PROMISE_IN_BOUNDS)

def _topk_body(scores_ref, vals_ref, idx_ref, buf, vv, vi):
    c = lax.axis_index("c")
    s = lax.axis_index("s")
    q = c * 16 + s

    pltpu.sync_copy(scores_ref.at[q], buf)

    neg = jnp.float32(-jnp.inf)
    lanes = lax.iota(jnp.int32, 16)
    big = jnp.int32(2**31 - 1)

    outv = jnp.zeros((16,), jnp.float32)
    outi = jnp.zeros((16,), jnp.int32)

    for kk in range(_TOPK):
        def body(i, carry):
            m, mi = carry
            v = buf[pl.ds(i * 16, 16)]
            upd = v > m
            m = jnp.where(upd, v, m)
            mi = jnp.where(upd, i, mi)
            return m, mi

        m, mi = lax.fori_loop(
            0, _CHUNKS, body,
            (jnp.full((16,), neg, jnp.float32), jnp.zeros((16,), jnp.int32)),
            unroll=8)
        mx = m
        for sh in (8, 4, 2, 1):
            mx = jnp.maximum(mx, _lane_permute(mx, lanes ^ sh))
        cand = jnp.where(m == mx, mi * 16 + lanes, big)
        pos = cand
        for sh in (8, 4, 2, 1):
            pos = jnp.minimum(pos, _lane_permute(pos, lanes ^ sh))
        outv = jnp.where(lanes == kk, mx, outv)
        outi = jnp.where(lanes == kk, pos, outi)
        plsc.store_scatter(buf, [pos], jnp.full((16,), neg, jnp.float32),
                           mask=lanes == 0)

    vv[...] = outv
    vi[...] = outi
    pltpu.sync_copy(vv, vals_ref.at[q])
    pltpu.sync_copy(vi, idx_ref.at[q])


def _topk_pass(scores_t):
    mesh = plsc.VectorSubcoreMesh(core_axis_name="c", subcore_axis_name="s")
    call = functools.partial(
        pl.kernel,
        out_type=[
            jax.ShapeDtypeStruct((_Q, 16), jnp.float32),
            jax.ShapeDtypeStruct((_Q, 16), jnp.int32),
        ],
        mesh=mesh,
        scratch_types=[
            pltpu.VMEM((_N,), jnp.float32),
            pltpu.VMEM((16,), jnp.float32),
            pltpu.VMEM((16,), jnp.int32),
        ],
        compiler_params=pltpu.CompilerParams(needs_layout_passes=False),
    )(_topk_body)
    return call(scores_t)


_NG = 3125
_GCH = 196


def _topk_body2(scores_ref, vals_ref, idx_ref, buf, gbuf, cval, cidx, vv, vi):
    c = lax.axis_index("c")
    s = lax.axis_index("s")
    q = c * 16 + s

    pltpu.sync_copy(scores_ref.at[q], buf)

    neg = jnp.float32(-jnp.inf)
    lanes = lax.iota(jnp.int32, 16)
    big = jnp.int32(2**31 - 1)

    def gbody(cc, carry):
        g0 = cc * 16
        m = jnp.full((16,), neg, jnp.float32)
        for j in range(16):
            m = jnp.maximum(m, buf[pl.ds(g0 + j * _NG, 16)])
        gbuf[pl.ds(g0, 16)] = m
        return carry

    lax.fori_loop(0, _GCH - 1, gbody, 0, unroll=4)
    g0 = (_GCH - 1) * 16
    m = jnp.full((16,), neg, jnp.float32)
    for j in range(15):
        m = jnp.maximum(m, buf[pl.ds(g0 + j * _NG, 16)])
    v15 = buf[pl.ds(_N - 16, 16)]
    shift = g0 + 15 * _NG - (_N - 16)
    m15 = _lane_permute(v15, jnp.minimum(lanes + shift, 15))
    m = jnp.maximum(m, m15)
    gbuf[pl.ds(g0, 16)] = jnp.where(g0 + lanes < _NG, m, neg)

    for kk in range(_TOPK):
        def body(i, carry):
            mm, mi = carry
            v = gbuf[pl.ds(i * 16, 16)]
            upd = v > mm
            mm = jnp.where(upd, v, mm)
            mi = jnp.where(upd, i, mi)
            return mm, mi

        mm, mi = lax.fori_loop(
            0, _GCH, body,
            (jnp.full((16,), neg, jnp.float32), jnp.zeros((16,), jnp.int32)),
            unroll=8)
        mx = mm
        for sh in (8, 4, 2, 1):
            mx = jnp.maximum(mx, _lane_permute(mx, lanes ^ sh))
        cand = jnp.where(mm == mx, mi * 16 + lanes, big)
        gsel = cand
        for sh in (8, 4, 2, 1):
            gsel = jnp.minimum(gsel, _lane_permute(gsel, lanes ^ sh))
        plsc.store_scatter(gbuf, [gsel], jnp.full((16,), neg, jnp.float32),
                           mask=lanes == 0)
        didx = gsel + _NG * lanes
        cval[pl.ds(kk * 16, 16)] = plsc.load_gather(buf, [didx])
        cidx[pl.ds(kk * 16, 16)] = didx

    outv = jnp.zeros((16,), jnp.float32)
    outi = jnp.zeros((16,), jnp.int32)
    for kk in range(_TOPK):
        mm = jnp.full((16,), neg, jnp.float32)
        mi = jnp.zeros((16,), jnp.int32)
        for i in range(_TOPK):
            v = cval[pl.ds(i * 16, 16)]
            upd = v > mm
            mm = jnp.where(upd, v, mm)
            mi = jnp.where(upd, i, mi)
        mx = mm
        for sh in (8, 4, 2, 1):
            mx = jnp.maximum(mx, _lane_permute(mx, lanes ^ sh))
        cand = jnp.where(mm == mx, mi * 16 + lanes, big)
        pos = cand
        for sh in (8, 4, 2, 1):
            pos = jnp.minimum(pos, _lane_permute(pos, lanes ^ sh))
        dsel = plsc.load_gather(cidx, [pos])
        outv = jnp.where(lanes == kk, mx, outv)
        outi = jnp.where(lanes == kk, dsel, outi)
        plsc.store_scatter(cval, [pos], jnp.full((16,), neg, jnp.float32),
                           mask=lanes == 0)

    vv[...] = outv
    vi[...] = outi
    pltpu.sync_copy(vv, vals_ref.at[q])
    pltpu.sync_copy(vi, idx_ref.at[q])


def _topk_pass2(scores_t):
    mesh = plsc.VectorSubcoreMesh(core_axis_name="c", subcore_axis_name="s")
    call = functools.partial(
        pl.kernel,
        out_type=[
            jax.ShapeDtypeStruct((_Q, 16), jnp.float32),
            jax.ShapeDtypeStruct((_Q, 16), jnp.int32),
        ],
        mesh=mesh,
        scratch_types=[
            pltpu.VMEM((_N,), jnp.float32),
            pltpu.VMEM((_GCH * 16,), jnp.float32),
            pltpu.VMEM((_TOPK * 16,), jnp.float32),
            pltpu.VMEM((_TOPK * 16,), jnp.int32),
            pltpu.VMEM((16,), jnp.float32),
            pltpu.VMEM((16,), jnp.int32),
        ],
        compiler_params=pltpu.CompilerParams(needs_layout_passes=False),
    )(_topk_body2)
    return call(scores_t)


def kernel(tf, doc_len, query_terms, k):
    doc_len = doc_len.astype(jnp.float32)
    tf = tf.astype(jnp.float32)

    counts = jnp.sum(
        jax.nn.one_hot(query_terms, _V, dtype=jnp.float32), axis=1)

    avg = jnp.mean(doc_len).reshape(1, 1)
    dl_row = doc_len.reshape(1, _N)

    cbs = [counts[:, c * _CB:c * _CB + _TBH[c]] for c in range(_NCB)]
    cbs[-1] = jnp.pad(cbs[-1], ((0, 0), (0, _CB - _TBH[-1])))
    counts_blocks = jnp.stack(cbs)

    scores_t = _fused_pass(tf.T, dl_row, counts_blocks, avg)

    vals_p, idx_p = _topk_pass2(scores_t)
    vals = vals_p[:, :_TOPK]
    idx = idx_p[:, :_TOPK]
    vals = vals + 0.0 * (jnp.asarray(k, jnp.float32) - float(_TOPK))
    return vals, idx


# speedup vs baseline: 2.9090x; 1.1265x over previous
import functools

import jax
import jax.numpy as jnp
from jax import lax
from jax.experimental import pallas as pl
from jax.experimental.pallas import tpu as pltpu
from jax.experimental.pallas import tpu_sc as plsc

_K1 = 1.5
_B = 0.75
_N = 50000
_V = 1000
_Q = 32
_L = 16
_TOPK = 10


_CB = 128
_NCB = -(-_V // _CB)
_TBH = [_CB] * (_NCB - 1) + [_V - _CB * (_NCB - 1)]
_RCH = 4096
_NRCH = _N // _RCH
_TAILR = _N - _NRCH * _RCH


def _fused_body(tft_hbm, dl_ref, cnt_ref, avg_ref, out_ref, bufa, bufb, sems):
    bufs = [bufa, bufb]

    def bcopy(c, b):
        h = _TBH[c]
        src = tft_hbm.at[pl.ds(c * _CB, h), :]
        dst = bufs[b] if h == _CB else bufs[b].at[pl.ds(0, h), :]
        return pltpu.make_async_copy(src, dst, sems.at[b])

    bcopy(0, 0).start()
    avg = avg_ref[0, 0]

    for c in range(_NCB):
        h = _TBH[c]
        cur = bufs[c % 2]
        bcopy(c, c % 2).wait()
        if c + 1 < _NCB:
            bcopy(c + 1, (c + 1) % 2).start()

        def df_chunk(i, d):
            blk = cur[0:h, pl.ds(i * _RCH, _RCH)]
            return d + jnp.sum((blk > 0).astype(jnp.float32), axis=1,
                               keepdims=True)

        df = lax.fori_loop(0, _NRCH, df_chunk,
                           jnp.zeros((h, 1), jnp.float32))
        blk = cur[0:h, pl.ds(_NRCH * _RCH, _TAILR)]
        df += jnp.sum((blk > 0).astype(jnp.float32), axis=1, keepdims=True)

        idf = jnp.log((_N - df + 0.5) / (df + 0.5))
        cnt = cnt_ref[c][:, 0:h]

        def sc_chunk(r0, rn):
            tfb = cur[0:h, pl.ds(r0, rn)]
            dlr = dl_ref[0:1, pl.ds(r0, rn)]
            norm = _K1 * (1.0 - _B + _B * dlr / avg)
            num = tfb * (_K1 + 1.0)
            ap = idf * num / (tfb + norm)
            return lax.dot_general(
                cnt, ap, (((1,), (0,)), ((), ())),
                preferred_element_type=jnp.float32,
                precision=lax.Precision.HIGHEST)

        if c == 0:
            def body0(i, _):
                r0 = i * _RCH
                out_ref[:, pl.ds(r0, _RCH)] = sc_chunk(r0, _RCH)
                return 0
            lax.fori_loop(0, _NRCH, body0, 0)
            out_ref[:, pl.ds(_NRCH * _RCH, _TAILR)] = sc_chunk(
                _NRCH * _RCH, _TAILR)
        else:
            def bodyn(i, _):
                r0 = i * _RCH
                out_ref[:, pl.ds(r0, _RCH)] += sc_chunk(r0, _RCH)
                return 0
            lax.fori_loop(0, _NRCH, bodyn, 0)
            out_ref[:, pl.ds(_NRCH * _RCH, _TAILR)] += sc_chunk(
                _NRCH * _RCH, _TAILR)


def _fused_pass(tft, dl_row, counts_blocks, avg):
    return pl.pallas_call(
        _fused_body,
        in_specs=[
            pl.BlockSpec(memory_space=pl.ANY),
            pl.BlockSpec(memory_space=pltpu.MemorySpace.VMEM),
            pl.BlockSpec(memory_space=pltpu.MemorySpace.VMEM),
            pl.BlockSpec(memory_space=pltpu.MemorySpace.VMEM),
        ],
        out_specs=pl.BlockSpec(memory_space=pltpu.MemorySpace.VMEM),
        out_shape=jax.ShapeDtypeStruct((_Q, _N), jnp.float32),
        scratch_shapes=[
            pltpu.VMEM((_CB, _N), jnp.float32),
            pltpu.VMEM((_CB, _N), jnp.float32),
            pltpu.SemaphoreType.DMA((2,)),
        ],
        compiler_params=pltpu.CompilerParams(
            vmem_limit_bytes=63 * 1024 * 1024),
    )(tft, dl_row, counts_blocks, avg)


_GATHER_DNUMS = lax.GatherDimensionNumbers(
    offset_dims=(), collapsed_slice_dims=(0,), start_index_map=(0,))


def _lane_permute(x, idx):
    return lax.gather(x, idx[:, None], _GATHER_DNUMS, slice_sizes=(1,),
                      mode=lax.GatherScatterMode.PROMISE_IN_BOUNDS)

_NG = 3125
_GCH = 196


def _topk_body2(scores_ref, vals_ref, idx_ref, buf, gbuf, cval, cidx, vv, vi):
    c = lax.axis_index("c")
    s = lax.axis_index("s")
    q = c * 16 + s

    pltpu.sync_copy(scores_ref.at[q], buf)

    neg = jnp.float32(-jnp.inf)
    lanes = lax.iota(jnp.int32, 16)
    big = jnp.int32(2**31 - 1)

    def gbody(cc, carry):
        g0 = cc * 16
        m = jnp.full((16,), neg, jnp.float32)
        for j in range(16):
            m = jnp.maximum(m, buf[pl.ds(g0 + j * _NG, 16)])
        gbuf[pl.ds(g0, 16)] = m
        return carry

    lax.fori_loop(0, _GCH - 1, gbody, 0, unroll=4)
    g0 = (_GCH - 1) * 16
    m = jnp.full((16,), neg, jnp.float32)
    for j in range(15):
        m = jnp.maximum(m, buf[pl.ds(g0 + j * _NG, 16)])
    v15 = buf[pl.ds(_N - 16, 16)]
    shift = g0 + 15 * _NG - (_N - 16)
    m15 = _lane_permute(v15, jnp.minimum(lanes + shift, 15))
    m = jnp.maximum(m, m15)
    gbuf[pl.ds(g0, 16)] = jnp.where(g0 + lanes < _NG, m, neg)

    for kk in range(_TOPK):
        def body(i, carry):
            mm, mi = carry
            v = gbuf[pl.ds(i * 16, 16)]
            upd = v > mm
            mm = jnp.where(upd, v, mm)
            mi = jnp.where(upd, i, mi)
            return mm, mi

        mm, mi = lax.fori_loop(
            0, _GCH, body,
            (jnp.full((16,), neg, jnp.float32), jnp.zeros((16,), jnp.int32)),
            unroll=8)
        mx = mm
        for sh in (8, 4, 2, 1):
            mx = jnp.maximum(mx, _lane_permute(mx, lanes ^ sh))
        cand = jnp.where(mm == mx, mi * 16 + lanes, big)
        gsel = cand
        for sh in (8, 4, 2, 1):
            gsel = jnp.minimum(gsel, _lane_permute(gsel, lanes ^ sh))
        plsc.store_scatter(gbuf, [gsel], jnp.full((16,), neg, jnp.float32),
                           mask=lanes == 0)
        didx = gsel + _NG * lanes
        cval[pl.ds(kk * 16, 16)] = plsc.load_gather(buf, [didx])
        cidx[pl.ds(kk * 16, 16)] = didx

    outv = jnp.zeros((16,), jnp.float32)
    outi = jnp.zeros((16,), jnp.int32)
    for kk in range(_TOPK):
        mm = jnp.full((16,), neg, jnp.float32)
        mi = jnp.zeros((16,), jnp.int32)
        for i in range(_TOPK):
            v = cval[pl.ds(i * 16, 16)]
            upd = v > mm
            mm = jnp.where(upd, v, mm)
            mi = jnp.where(upd, i, mi)
        mx = mm
        for sh in (8, 4, 2, 1):
            mx = jnp.maximum(mx, _lane_permute(mx, lanes ^ sh))
        cand = jnp.where(mm == mx, mi * 16 + lanes, big)
        pos = cand
        for sh in (8, 4, 2, 1):
            pos = jnp.minimum(pos, _lane_permute(pos, lanes ^ sh))
        dsel = plsc.load_gather(cidx, [pos])
        outv = jnp.where(lanes == kk, mx, outv)
        outi = jnp.where(lanes == kk, dsel, outi)
        plsc.store_scatter(cval, [pos], jnp.full((16,), neg, jnp.float32),
                           mask=lanes == 0)

    vv[...] = outv
    vi[...] = outi
    pltpu.sync_copy(vv, vals_ref.at[q])
    pltpu.sync_copy(vi, idx_ref.at[q])


def _topk_pass2(scores_t):
    mesh = plsc.VectorSubcoreMesh(core_axis_name="c", subcore_axis_name="s")
    call = functools.partial(
        pl.kernel,
        out_type=[
            jax.ShapeDtypeStruct((_Q, 16), jnp.float32),
            jax.ShapeDtypeStruct((_Q, 16), jnp.int32),
        ],
        mesh=mesh,
        scratch_types=[
            pltpu.VMEM((_N,), jnp.float32),
            pltpu.VMEM((_GCH * 16,), jnp.float32),
            pltpu.VMEM((_TOPK * 16,), jnp.float32),
            pltpu.VMEM((_TOPK * 16,), jnp.int32),
            pltpu.VMEM((16,), jnp.float32),
            pltpu.VMEM((16,), jnp.int32),
        ],
        compiler_params=pltpu.CompilerParams(needs_layout_passes=False),
    )(_topk_body2)
    return call(scores_t)


def kernel(tf, doc_len, query_terms, k):
    doc_len = doc_len.astype(jnp.float32)
    tf = tf.astype(jnp.float32)

    counts = jnp.sum(
        jax.nn.one_hot(query_terms, _V, dtype=jnp.float32), axis=1)

    avg = jnp.mean(doc_len).reshape(1, 1)
    dl_row = doc_len.reshape(1, _N)

    cbs = [counts[:, c * _CB:c * _CB + _TBH[c]] for c in range(_NCB)]
    cbs[-1] = jnp.pad(cbs[-1], ((0, 0), (0, _CB - _TBH[-1])))
    counts_blocks = jnp.stack(cbs)

    scores_t = _fused_pass(tf.T, dl_row, counts_blocks, avg)

    vals_p, idx_p = _topk_pass2(scores_t)
    vals = vals_p[:, :_TOPK]
    idx = idx_p[:, :_TOPK]
    vals = vals + 0.0 * (jnp.asarray(k, jnp.float32) - float(_TOPK))
    return vals, idx


# speedup vs baseline: 3.0960x; 1.0643x over previous
import functools

import jax
import jax.numpy as jnp
from jax import lax
from jax.experimental import pallas as pl
from jax.experimental.pallas import tpu as pltpu
from jax.experimental.pallas import tpu_sc as plsc

_K1 = 1.5
_B = 0.75
_N = 50000
_V = 1000
_Q = 32
_L = 16
_TOPK = 10


_CB = 128
_NCB = -(-_V // _CB)
_TBH = [_CB] * (_NCB - 1) + [_V - _CB * (_NCB - 1)]
_RCH = 8192
_NRCH = _N // _RCH
_TAILR = _N - _NRCH * _RCH


def _fused_body(tft_hbm, dl_ref, cnt_ref, avg_ref, out_ref, bufa, bufb, sems):
    bufs = [bufa, bufb]

    def bcopy(c, b):
        h = _TBH[c]
        src = tft_hbm.at[pl.ds(c * _CB, h), :]
        dst = bufs[b] if h == _CB else bufs[b].at[pl.ds(0, h), :]
        return pltpu.make_async_copy(src, dst, sems.at[b])

    bcopy(0, 0).start()
    avg = avg_ref[0, 0]

    for c in range(_NCB):
        h = _TBH[c]
        cur = bufs[c % 2]
        bcopy(c, c % 2).wait()
        if c + 1 < _NCB:
            bcopy(c + 1, (c + 1) % 2).start()

        def df_chunk(i, d):
            blk = cur[0:h, pl.ds(i * _RCH, _RCH)]
            return d + jnp.sum((blk > 0).astype(jnp.float32), axis=1,
                               keepdims=True)

        df = lax.fori_loop(0, _NRCH, df_chunk,
                           jnp.zeros((h, 1), jnp.float32))
        blk = cur[0:h, pl.ds(_NRCH * _RCH, _TAILR)]
        df += jnp.sum((blk > 0).astype(jnp.float32), axis=1, keepdims=True)

        idf = jnp.log((_N - df + 0.5) / (df + 0.5))
        cnt = cnt_ref[c][:, 0:h]

        def sc_chunk(r0, rn):
            tfb = cur[0:h, pl.ds(r0, rn)]
            dlr = dl_ref[0:1, pl.ds(r0, rn)]
            norm = _K1 * (1.0 - _B + _B * dlr / avg)
            num = tfb * (_K1 + 1.0)
            ap = idf * num / (tfb + norm)
            return lax.dot_general(
                cnt, ap, (((1,), (0,)), ((), ())),
                preferred_element_type=jnp.float32,
                precision=lax.Precision.HIGHEST)

        if c == 0:
            def body0(i, _):
                r0 = i * _RCH
                out_ref[:, pl.ds(r0, _RCH)] = sc_chunk(r0, _RCH)
                return 0
            lax.fori_loop(0, _NRCH, body0, 0)
            out_ref[:, pl.ds(_NRCH * _RCH, _TAILR)] = sc_chunk(
                _NRCH * _RCH, _TAILR)
        else:
            def bodyn(i, _):
                r0 = i * _RCH
                out_ref[:, pl.ds(r0, _RCH)] += sc_chunk(r0, _RCH)
                return 0
            lax.fori_loop(0, _NRCH, bodyn, 0)
            out_ref[:, pl.ds(_NRCH * _RCH, _TAILR)] += sc_chunk(
                _NRCH * _RCH, _TAILR)


def _fused_pass(tft, dl_row, counts_blocks, avg):
    return pl.pallas_call(
        _fused_body,
        in_specs=[
            pl.BlockSpec(memory_space=pl.ANY),
            pl.BlockSpec(memory_space=pltpu.MemorySpace.VMEM),
            pl.BlockSpec(memory_space=pltpu.MemorySpace.VMEM),
            pl.BlockSpec(memory_space=pltpu.MemorySpace.VMEM),
        ],
        out_specs=pl.BlockSpec(memory_space=pltpu.MemorySpace.VMEM),
        out_shape=jax.ShapeDtypeStruct((_Q, _N), jnp.float32),
        scratch_shapes=[
            pltpu.VMEM((_CB, _N), jnp.float32),
            pltpu.VMEM((_CB, _N), jnp.float32),
            pltpu.SemaphoreType.DMA((2,)),
        ],
        compiler_params=pltpu.CompilerParams(
            vmem_limit_bytes=63 * 1024 * 1024),
    )(tft, dl_row, counts_blocks, avg)


_GATHER_DNUMS = lax.GatherDimensionNumbers(
    offset_dims=(), collapsed_slice_dims=(0,), start_index_map=(0,))


def _lane_permute(x, idx):
    return lax.gather(x, idx[:, None], _GATHER_DNUMS, slice_sizes=(1,),
                      mode=lax.GatherScatterMode.PROMISE_IN_BOUNDS)

_NG = 3125
_GCH = 196


def _topk_body2(scores_ref, vals_ref, idx_ref, buf, gbuf, cval, cidx, vv, vi):
    c = lax.axis_index("c")
    s = lax.axis_index("s")
    q = c * 16 + s

    pltpu.sync_copy(scores_ref.at[q], buf)

    neg = jnp.float32(-jnp.inf)
    lanes = lax.iota(jnp.int32, 16)
    big = jnp.int32(2**31 - 1)

    def gbody(cc, carry):
        g0 = cc * 16
        m = jnp.full((16,), neg, jnp.float32)
        for j in range(16):
            m = jnp.maximum(m, buf[pl.ds(g0 + j * _NG, 16)])
        gbuf[pl.ds(g0, 16)] = m
        return carry

    lax.fori_loop(0, _GCH - 1, gbody, 0, unroll=4)
    g0 = (_GCH - 1) * 16
    m = jnp.full((16,), neg, jnp.float32)
    for j in range(15):
        m = jnp.maximum(m, buf[pl.ds(g0 + j * _NG, 16)])
    v15 = buf[pl.ds(_N - 16, 16)]
    shift = g0 + 15 * _NG - (_N - 16)
    m15 = _lane_permute(v15, jnp.minimum(lanes + shift, 15))
    m = jnp.maximum(m, m15)
    gbuf[pl.ds(g0, 16)] = jnp.where(g0 + lanes < _NG, m, neg)

    for kk in range(_TOPK):
        def body(i, carry):
            mm, mi = carry
            v = gbuf[pl.ds(i * 16, 16)]
            upd = v > mm
            mm = jnp.where(upd, v, mm)
            mi = jnp.where(upd, i, mi)
            return mm, mi

        mm, mi = lax.fori_loop(
            0, _GCH, body,
            (jnp.full((16,), neg, jnp.float32), jnp.zeros((16,), jnp.int32)),
            unroll=8)
        mx = mm
        for sh in (8, 4, 2, 1):
            mx = jnp.maximum(mx, _lane_permute(mx, lanes ^ sh))
        cand = jnp.where(mm == mx, mi * 16 + lanes, big)
        gsel = cand
        for sh in (8, 4, 2, 1):
            gsel = jnp.minimum(gsel, _lane_permute(gsel, lanes ^ sh))
        plsc.store_scatter(gbuf, [gsel], jnp.full((16,), neg, jnp.float32),
                           mask=lanes == 0)
        didx = gsel + _NG * lanes
        cval[pl.ds(kk * 16, 16)] = plsc.load_gather(buf, [didx])
        cidx[pl.ds(kk * 16, 16)] = didx

    outv = jnp.zeros((16,), jnp.float32)
    outi = jnp.zeros((16,), jnp.int32)
    for kk in range(_TOPK):
        mm = jnp.full((16,), neg, jnp.float32)
        mi = jnp.zeros((16,), jnp.int32)
        for i in range(_TOPK):
            v = cval[pl.ds(i * 16, 16)]
            upd = v > mm
            mm = jnp.where(upd, v, mm)
            mi = jnp.where(upd, i, mi)
        mx = mm
        for sh in (8, 4, 2, 1):
            mx = jnp.maximum(mx, _lane_permute(mx, lanes ^ sh))
        cand = jnp.where(mm == mx, mi * 16 + lanes, big)
        pos = cand
        for sh in (8, 4, 2, 1):
            pos = jnp.minimum(pos, _lane_permute(pos, lanes ^ sh))
        dsel = plsc.load_gather(cidx, [pos])
        outv = jnp.where(lanes == kk, mx, outv)
        outi = jnp.where(lanes == kk, dsel, outi)
        plsc.store_scatter(cval, [pos], jnp.full((16,), neg, jnp.float32),
                           mask=lanes == 0)

    vv[...] = outv
    vi[...] = outi
    pltpu.sync_copy(vv, vals_ref.at[q])
    pltpu.sync_copy(vi, idx_ref.at[q])


def _topk_pass2(scores_t):
    mesh = plsc.VectorSubcoreMesh(core_axis_name="c", subcore_axis_name="s")
    call = functools.partial(
        pl.kernel,
        out_type=[
            jax.ShapeDtypeStruct((_Q, 16), jnp.float32),
            jax.ShapeDtypeStruct((_Q, 16), jnp.int32),
        ],
        mesh=mesh,
        scratch_types=[
            pltpu.VMEM((_N,), jnp.float32),
            pltpu.VMEM((_GCH * 16,), jnp.float32),
            pltpu.VMEM((_TOPK * 16,), jnp.float32),
            pltpu.VMEM((_TOPK * 16,), jnp.int32),
            pltpu.VMEM((16,), jnp.float32),
            pltpu.VMEM((16,), jnp.int32),
        ],
        compiler_params=pltpu.CompilerParams(needs_layout_passes=False),
    )(_topk_body2)
    return call(scores_t)


def kernel(tf, doc_len, query_terms, k):
    doc_len = doc_len.astype(jnp.float32)
    tf = tf.astype(jnp.float32)

    counts = jnp.sum(
        jax.nn.one_hot(query_terms, _V, dtype=jnp.float32), axis=1)

    avg = jnp.mean(doc_len).reshape(1, 1)
    dl_row = doc_len.reshape(1, _N)

    cbs = [counts[:, c * _CB:c * _CB + _TBH[c]] for c in range(_NCB)]
    cbs[-1] = jnp.pad(cbs[-1], ((0, 0), (0, _CB - _TBH[-1])))
    counts_blocks = jnp.stack(cbs)

    scores_t = _fused_pass(tf.T, dl_row, counts_blocks, avg)

    vals_p, idx_p = _topk_pass2(scores_t)
    vals = vals_p[:, :_TOPK]
    idx = idx_p[:, :_TOPK]
    vals = vals + 0.0 * (jnp.asarray(k, jnp.float32) - float(_TOPK))
    return vals, idx


# speedup vs baseline: 3.1918x; 1.0309x over previous
import functools

import jax
import jax.numpy as jnp
from jax import lax
from jax.experimental import pallas as pl
from jax.experimental.pallas import tpu as pltpu
from jax.experimental.pallas import tpu_sc as plsc

_K1 = 1.5
_B = 0.75
_N = 50000
_V = 1000
_Q = 32
_L = 16
_TOPK = 10


_CB = 128
_NCB = -(-_V // _CB)
_TBH = [_CB] * (_NCB - 1) + [_V - _CB * (_NCB - 1)]
_RCH = 16384
_NRCH = _N // _RCH
_TAILR = _N - _NRCH * _RCH


def _fused_body(tft_hbm, dl_ref, cnt_ref, avg_ref, out_ref, bufa, bufb, sems):
    bufs = [bufa, bufb]

    def bcopy(c, b):
        h = _TBH[c]
        src = tft_hbm.at[pl.ds(c * _CB, h), :]
        dst = bufs[b] if h == _CB else bufs[b].at[pl.ds(0, h), :]
        return pltpu.make_async_copy(src, dst, sems.at[b])

    bcopy(0, 0).start()
    avg = avg_ref[0, 0]

    for c in range(_NCB):
        h = _TBH[c]
        cur = bufs[c % 2]
        bcopy(c, c % 2).wait()
        if c + 1 < _NCB:
            bcopy(c + 1, (c + 1) % 2).start()

        def df_chunk(i, d):
            blk = cur[0:h, pl.ds(i * _RCH, _RCH)]
            return d + jnp.sum((blk > 0).astype(jnp.float32), axis=1,
                               keepdims=True)

        df = lax.fori_loop(0, _NRCH, df_chunk,
                           jnp.zeros((h, 1), jnp.float32))
        blk = cur[0:h, pl.ds(_NRCH * _RCH, _TAILR)]
        df += jnp.sum((blk > 0).astype(jnp.float32), axis=1, keepdims=True)

        idf = jnp.log((_N - df + 0.5) / (df + 0.5))
        cnt = cnt_ref[c][:, 0:h]

        def sc_chunk(r0, rn):
            tfb = cur[0:h, pl.ds(r0, rn)]
            dlr = dl_ref[0:1, pl.ds(r0, rn)]
            norm = _K1 * (1.0 - _B + _B * dlr / avg)
            num = tfb * (_K1 + 1.0)
            ap = idf * num / (tfb + norm)
            return lax.dot_general(
                cnt, ap, (((1,), (0,)), ((), ())),
                preferred_element_type=jnp.float32,
                precision=lax.Precision.HIGHEST)

        if c == 0:
            def body0(i, _):
                r0 = i * _RCH
                out_ref[:, pl.ds(r0, _RCH)] = sc_chunk(r0, _RCH)
                return 0
            lax.fori_loop(0, _NRCH, body0, 0)
            out_ref[:, pl.ds(_NRCH * _RCH, _TAILR)] = sc_chunk(
                _NRCH * _RCH, _TAILR)
        else:
            def bodyn(i, _):
                r0 = i * _RCH
                out_ref[:, pl.ds(r0, _RCH)] += sc_chunk(r0, _RCH)
                return 0
            lax.fori_loop(0, _NRCH, bodyn, 0)
            out_ref[:, pl.ds(_NRCH * _RCH, _TAILR)] += sc_chunk(
                _NRCH * _RCH, _TAILR)


def _fused_pass(tft, dl_row, counts_blocks, avg):
    return pl.pallas_call(
        _fused_body,
        in_specs=[
            pl.BlockSpec(memory_space=pl.ANY),
            pl.BlockSpec(memory_space=pltpu.MemorySpace.VMEM),
            pl.BlockSpec(memory_space=pltpu.MemorySpace.VMEM),
            pl.BlockSpec(memory_space=pltpu.MemorySpace.VMEM),
        ],
        out_specs=pl.BlockSpec(memory_space=pltpu.MemorySpace.VMEM),
        out_shape=jax.ShapeDtypeStruct((_Q, _N), jnp.float32),
        scratch_shapes=[
            pltpu.VMEM((_CB, _N), jnp.float32),
            pltpu.VMEM((_CB, _N), jnp.float32),
            pltpu.SemaphoreType.DMA((2,)),
        ],
        compiler_params=pltpu.CompilerParams(
            vmem_limit_bytes=63 * 1024 * 1024),
    )(tft, dl_row, counts_blocks, avg)


_GATHER_DNUMS = lax.GatherDimensionNumbers(
    offset_dims=(), collapsed_slice_dims=(0,), start_index_map=(0,))


def _lane_permute(x, idx):
    return lax.gather(x, idx[:, None], _GATHER_DNUMS, slice_sizes=(1,),
                      mode=lax.GatherScatterMode.PROMISE_IN_BOUNDS)

_NG = 3125
_GCH = 196


def _topk_body2(scores_ref, vals_ref, idx_ref, buf, gbuf, cval, cidx, vv, vi):
    c = lax.axis_index("c")
    s = lax.axis_index("s")
    q = c * 16 + s

    pltpu.sync_copy(scores_ref.at[q], buf)

    neg = jnp.float32(-jnp.inf)
    lanes = lax.iota(jnp.int32, 16)
    big = jnp.int32(2**31 - 1)

    def gbody(cc, carry):
        g0 = cc * 16
        m = jnp.full((16,), neg, jnp.float32)
        for j in range(16):
            m = jnp.maximum(m, buf[pl.ds(g0 + j * _NG, 16)])
        gbuf[pl.ds(g0, 16)] = m
        return carry

    lax.fori_loop(0, _GCH - 1, gbody, 0, unroll=4)
    g0 = (_GCH - 1) * 16
    m = jnp.full((16,), neg, jnp.float32)
    for j in range(15):
        m = jnp.maximum(m, buf[pl.ds(g0 + j * _NG, 16)])
    v15 = buf[pl.ds(_N - 16, 16)]
    shift = g0 + 15 * _NG - (_N - 16)
    m15 = _lane_permute(v15, jnp.minimum(lanes + shift, 15))
    m = jnp.maximum(m, m15)
    gbuf[pl.ds(g0, 16)] = jnp.where(g0 + lanes < _NG, m, neg)

    for kk in range(_TOPK):
        def body(i, carry):
            mm, mi = carry
            v = gbuf[pl.ds(i * 16, 16)]
            upd = v > mm
            mm = jnp.where(upd, v, mm)
            mi = jnp.where(upd, i, mi)
            return mm, mi

        mm, mi = lax.fori_loop(
            0, _GCH, body,
            (jnp.full((16,), neg, jnp.float32), jnp.zeros((16,), jnp.int32)),
            unroll=8)
        mx = mm
        for sh in (8, 4, 2, 1):
            mx = jnp.maximum(mx, _lane_permute(mx, lanes ^ sh))
        cand = jnp.where(mm == mx, mi * 16 + lanes, big)
        gsel = cand
        for sh in (8, 4, 2, 1):
            gsel = jnp.minimum(gsel, _lane_permute(gsel, lanes ^ sh))
        plsc.store_scatter(gbuf, [gsel], jnp.full((16,), neg, jnp.float32),
                           mask=lanes == 0)
        didx = gsel + _NG * lanes
        cval[pl.ds(kk * 16, 16)] = plsc.load_gather(buf, [didx])
        cidx[pl.ds(kk * 16, 16)] = didx

    outv = jnp.zeros((16,), jnp.float32)
    outi = jnp.zeros((16,), jnp.int32)
    for kk in range(_TOPK):
        mm = jnp.full((16,), neg, jnp.float32)
        mi = jnp.zeros((16,), jnp.int32)
        for i in range(_TOPK):
            v = cval[pl.ds(i * 16, 16)]
            upd = v > mm
            mm = jnp.where(upd, v, mm)
            mi = jnp.where(upd, i, mi)
        mx = mm
        for sh in (8, 4, 2, 1):
            mx = jnp.maximum(mx, _lane_permute(mx, lanes ^ sh))
        cand = jnp.where(mm == mx, mi * 16 + lanes, big)
        pos = cand
        for sh in (8, 4, 2, 1):
            pos = jnp.minimum(pos, _lane_permute(pos, lanes ^ sh))
        dsel = plsc.load_gather(cidx, [pos])
        outv = jnp.where(lanes == kk, mx, outv)
        outi = jnp.where(lanes == kk, dsel, outi)
        plsc.store_scatter(cval, [pos], jnp.full((16,), neg, jnp.float32),
                           mask=lanes == 0)

    vv[...] = outv
    vi[...] = outi
    pltpu.sync_copy(vv, vals_ref.at[q])
    pltpu.sync_copy(vi, idx_ref.at[q])


def _topk_pass2(scores_t):
    mesh = plsc.VectorSubcoreMesh(core_axis_name="c", subcore_axis_name="s")
    call = functools.partial(
        pl.kernel,
        out_type=[
            jax.ShapeDtypeStruct((_Q, 16), jnp.float32),
            jax.ShapeDtypeStruct((_Q, 16), jnp.int32),
        ],
        mesh=mesh,
        scratch_types=[
            pltpu.VMEM((_N,), jnp.float32),
            pltpu.VMEM((_GCH * 16,), jnp.float32),
            pltpu.VMEM((_TOPK * 16,), jnp.float32),
            pltpu.VMEM((_TOPK * 16,), jnp.int32),
            pltpu.VMEM((16,), jnp.float32),
            pltpu.VMEM((16,), jnp.int32),
        ],
        compiler_params=pltpu.CompilerParams(needs_layout_passes=False),
    )(_topk_body2)
    return call(scores_t)


def kernel(tf, doc_len, query_terms, k):
    doc_len = doc_len.astype(jnp.float32)
    tf = tf.astype(jnp.float32)

    counts = jnp.sum(
        jax.nn.one_hot(query_terms, _V, dtype=jnp.float32), axis=1)

    avg = jnp.mean(doc_len).reshape(1, 1)
    dl_row = doc_len.reshape(1, _N)

    cbs = [counts[:, c * _CB:c * _CB + _TBH[c]] for c in range(_NCB)]
    cbs[-1] = jnp.pad(cbs[-1], ((0, 0), (0, _CB - _TBH[-1])))
    counts_blocks = jnp.stack(cbs)

    scores_t = _fused_pass(tf.T, dl_row, counts_blocks, avg)

    vals_p, idx_p = _topk_pass2(scores_t)
    vals = vals_p[:, :_TOPK]
    idx = idx_p[:, :_TOPK]
    vals = vals + 0.0 * (jnp.asarray(k, jnp.float32) - float(_TOPK))
    return vals, idx
